# baseline (speedup 1.0000x reference)
import functools

import jax
import jax.numpy as jnp
from jax import lax
from jax.experimental import pallas as pl
from jax.experimental.pallas import tpu as pltpu
from jax.experimental.pallas import tpu_sc as plsc

NCU = 2
NS = 16
NW = NCU * NS

CK = 128
CPB = 20


def _mm_body(x_ref, w_ref, o_ref):
    o_ref[...] = lax.dot_general(x_ref[...], w_ref[...],
                                 (((1,), (1,)), ((), ())),
                                 preferred_element_type=jnp.float32)


def _combine_body(p0_ref, p1_ref, d0_ref, d1_ref, x_ref, w_ref, b_ref, o_ref):
    s = p0_ref[...] + p1_ref[...]
    deg = d0_ref[...] + d1_ref[...]
    inv = 1.0 / jnp.maximum(deg, 1.0)
    mm = lax.dot_general(x_ref[...], w_ref[...], (((1,), (1,)), ((), ())),
                         preferred_element_type=jnp.float32)
    o_ref[...] = s * inv + mm + b_ref[...]


def _make_sc_scatter(n_nodes, n_edges, d):
    epw = n_edges // NW
    nb = epw // (CPB * CK)
    n_pad = -(-n_nodes // (NS * 8)) * (NS * 8)
    rpt = n_pad // NS
    dr = -(-n_pad // (128 * 16)) * 16
    assert epw % (CPB * CK) == 0 and CK % 16 == 0

    mesh = plsc.VectorSubcoreMesh(core_axis_name="c", subcore_axis_name="s",
                                  num_cores=NCU)

    @functools.partial(
        pl.kernel,
        out_type=(jax.ShapeDtypeStruct((n_pad, d), jnp.float32),
                  jax.ShapeDtypeStruct((n_pad, d), jnp.float32),
                  jax.ShapeDtypeStruct((dr, 128), jnp.float32),
                  jax.ShapeDtypeStruct((dr, 128), jnp.float32)),
        mesh=mesh,
        scratch_types=[
            pltpu.VMEM((CPB, CK), jnp.int32),
            pltpu.VMEM((CPB, CK), jnp.int32),
            pltpu.VMEM((2, CK, d), jnp.float32),
            pltpu.VMEM((dr, 128), jnp.float32),
            pltpu.VMEM((dr,), jnp.int32),
            pltpu.VMEM_SHARED((n_pad, d), jnp.float32),
            pltpu.VMEM_SHARED((dr, 128), jnp.float32),
            pltpu.SemaphoreType.DMA,
            pltpu.SemaphoreType.DMA,
            pltpu.SemaphoreType.DMA,
            pltpu.SemaphoreType.DMA,
            pltpu.SemaphoreType.DMA,
            pltpu.SemaphoreType.DMA,
        ],
        compiler_params=pltpu.CompilerParams(needs_layout_passes=False),
    )
    def sc_scatter(y_hbm, src_hbm, dst_hbm, zero_hbm,
                   out0_hbm, out1_hbm, deg0_hbm, deg1_hbm,
                   src_v, dst_v, rows_v, deg_v, rowidx, acc, dacc,
                   sem_g0, sem_g1, sem_g2, sem_s0, sem_s1, sem_s2):
        sem_g = (sem_g0, sem_g1, sem_g2)
        sem_s = (sem_s0, sem_s1, sem_s2)
        cid = lax.axis_index("c")
        sid = lax.axis_index("s")
        wid = sid * NCU + cid

        z16 = jnp.zeros((16,), jnp.float32)

        def zdeg(i, carry):
            for j in range(128 // 16):
                deg_v[i, pl.ds(j * 16, 16)] = z16
            return carry

        lax.fori_loop(0, dr, zdeg, 0)
        for m in range(dr // 16):
            rowidx[pl.ds(m * 16, 16)] = lax.iota(jnp.int32, 16) + m * 16

        pltpu.sync_copy(zero_hbm.at[pl.ds(0, rpt)], acc.at[pl.ds(sid * rpt, rpt)])

        @pl.when(sid < dr // 8)
        def _():
            pltpu.sync_copy(zero_hbm.at[pl.ds(0, 8)], dacc.at[pl.ds(sid * 8, 8)])

        plsc.subcore_barrier()

        def block(bi, carry):
            pltpu.sync_copy(src_hbm.at[wid * nb + bi], src_v)
            pltpu.sync_copy(dst_hbm.at[wid * nb + bi], dst_v)

            nbuf = 2
            gathers = [None] * nbuf
            scatters = [None] * nbuf
            for j0 in range(nbuf - 1):
                gathers[j0] = pltpu.async_copy(y_hbm.at[src_v.at[j0]],
                                               rows_v.at[j0], sem_g[j0])
            for j in range(CPB):
                p = j % nbuf
                q = (j + nbuf - 1) % nbuf
                if j + nbuf - 1 < CPB:
                    if scatters[q] is not None:
                        scatters[q].wait()
                        scatters[q] = None
                    gathers[q] = pltpu.async_copy(
                        y_hbm.at[src_v.at[j + nbuf - 1]],
                        rows_v.at[q], sem_g[q])
                gathers[p].wait()
                scatters[p] = pltpu.async_copy(rows_v.at[p],
                                               acc.at[dst_v.at[j]],
                                               sem_s[p], add=True)
                for k in range(CK // 16):
                    vec = dst_v[j, pl.ds(k * 16, 16)]
                    cnt, last = plsc.scan_count(vec)
                    plsc.addupdate_scatter(deg_v,
                                           [lax.shift_right_logical(vec, 7),
                                            lax.bitwise_and(vec, 127)],
                                           cnt.astype(jnp.float32), mask=last)
            for s in scatters:
                if s is not None:
                    s.wait()
            return carry

        lax.fori_loop(0, nb, block, 0)

        pltpu.sync_copy(deg_v, dacc.at[rowidx], add=True)
        plsc.subcore_barrier()

        @pl.when(cid == 0)
        def _():
            pltpu.sync_copy(acc.at[pl.ds(sid * rpt, rpt)],
                            out0_hbm.at[pl.ds(sid * rpt, rpt)])

            @pl.when(sid < dr // 8)
            def _():
                pltpu.sync_copy(dacc.at[pl.ds(sid * 8, 8)],
                                deg0_hbm.at[pl.ds(sid * 8, 8)])

        @pl.when(cid == 1)
        def _():
            pltpu.sync_copy(acc.at[pl.ds(sid * rpt, rpt)],
                            out1_hbm.at[pl.ds(sid * rpt, rpt)])

            @pl.when(sid < dr // 8)
            def _():
                pltpu.sync_copy(dacc.at[pl.ds(sid * 8, 8)],
                                deg1_hbm.at[pl.ds(sid * 8, 8)])

    return sc_scatter, n_pad, dr


def kernel(x, edge_index, W_l, W_r, b):
    n, d = x.shape
    e = edge_index.shape[1]
    blk = 1000

    y = pl.pallas_call(
        _mm_body,
        grid=(n // blk,),
        in_specs=[pl.BlockSpec((blk, d), lambda i: (i, 0)),
                  pl.BlockSpec((d, d), lambda i: (0, 0))],
        out_specs=pl.BlockSpec((blk, d), lambda i: (i, 0)),
        out_shape=jax.ShapeDtypeStruct((n, d), jnp.float32),
    )(x, W_l)

    bsz = CPB * CK
    epw = -(-(e // NW) // bsz) * bsz
    sc_scatter, n_pad, dr = _make_sc_scatter(n, NW * epw, d)
    pad = NW * epw - e
    src_flat, dst_flat = edge_index[0], edge_index[1]
    if pad:
        ar = jnp.arange(pad, dtype=jnp.int32)
        src_flat = jnp.concatenate([src_flat, (ar * 977) % n])
        dst_flat = jnp.concatenate([dst_flat, n + ar % (n_pad - n)])
    zeros = jnp.zeros((n_pad // NS, d), jnp.float32)
    src3 = src_flat.reshape(-1, CPB, CK)
    dst3 = dst_flat.reshape(-1, CPB, CK)
    acc0, acc1, deg0, deg1 = sc_scatter(y, src3, dst3, zeros)
    d0 = deg0.reshape(dr * 128)[:n].reshape(n, 1)
    d1 = deg1.reshape(dr * 128)[:n].reshape(n, 1)

    out = pl.pallas_call(
        _combine_body,
        grid=(n // blk,),
        in_specs=[pl.BlockSpec((blk, d), lambda i: (i, 0)),
                  pl.BlockSpec((blk, d), lambda i: (i, 0)),
                  pl.BlockSpec((blk, 1), lambda i: (i, 0)),
                  pl.BlockSpec((blk, 1), lambda i: (i, 0)),
                  pl.BlockSpec((blk, d), lambda i: (i, 0)),
                  pl.BlockSpec((d, d), lambda i: (0, 0)),
                  pl.BlockSpec((1, d), lambda i: (0, 0))],
        out_specs=pl.BlockSpec((blk, d), lambda i: (i, 0)),
        out_shape=jax.ShapeDtypeStruct((n, d), jnp.float32),
    )(acc0, acc1, d0, d1, x, W_r, b.reshape(1, d))

    return out


# speedup vs baseline: 1.1671x; 1.1671x over previous
import functools

import jax
import jax.numpy as jnp
from jax import lax
from jax.experimental import pallas as pl
from jax.experimental.pallas import tpu as pltpu
from jax.experimental.pallas import tpu_sc as plsc

NCU = 2
NS = 16
NW = NCU * NS

CK = 128
CPB = 20


def _combine_body(p0_ref, p1_ref, d0_ref, d1_ref, x_ref, wl_ref, wr_ref,
                  b_ref, o_ref):
    s = p0_ref[...] + p1_ref[...]
    deg = d0_ref[...] + d1_ref[...]
    inv = 1.0 / jnp.maximum(deg, 1.0)
    blk, dd = s.shape
    s3 = s.reshape(blk // 128, 128, dd)
    mean = (s3 * inv[:, :, None]).reshape(blk, dd)
    mm_l = lax.dot_general(mean, wl_ref[...], (((1,), (1,)), ((), ())),
                           preferred_element_type=jnp.float32)
    mm_r = lax.dot_general(x_ref[...], wr_ref[...], (((1,), (1,)), ((), ())),
                           preferred_element_type=jnp.float32)
    o_ref[...] = mm_l + mm_r + b_ref[...]


def _make_sc_scatter(n_nodes, n_edges, d):
    epw = n_edges // NW
    nb = epw // (CPB * CK)
    n_pad = -(-n_nodes // (NS * 8)) * (NS * 8)
    rpt = n_pad // NS
    dr = -(-n_pad // (128 * 16)) * 16
    assert epw % (CPB * CK) == 0 and CK % 16 == 0

    mesh = plsc.VectorSubcoreMesh(core_axis_name="c", subcore_axis_name="s",
                                  num_cores=NCU)

    @functools.partial(
        pl.kernel,
        out_type=(jax.ShapeDtypeStruct((n_pad, d), jnp.float32),
                  jax.ShapeDtypeStruct((n_pad, d), jnp.float32),
                  jax.ShapeDtypeStruct((dr, 128), jnp.float32),
                  jax.ShapeDtypeStruct((dr, 128), jnp.float32)),
        mesh=mesh,
        scratch_types=[
            pltpu.VMEM((CPB, CK), jnp.int32),
            pltpu.VMEM((CPB, CK), jnp.int32),
            pltpu.VMEM((2, CK, d), jnp.float32),
            pltpu.VMEM((dr, 128), jnp.float32),
            pltpu.VMEM((dr,), jnp.int32),
            pltpu.VMEM_SHARED((n_pad, d), jnp.float32),
            pltpu.VMEM_SHARED((dr, 128), jnp.float32),
            pltpu.SemaphoreType.DMA,
            pltpu.SemaphoreType.DMA,
            pltpu.SemaphoreType.DMA,
            pltpu.SemaphoreType.DMA,
            pltpu.SemaphoreType.DMA,
            pltpu.SemaphoreType.DMA,
        ],
        compiler_params=pltpu.CompilerParams(needs_layout_passes=False),
    )
    def sc_scatter(y_hbm, edge_hbm, zero_hbm,
                   out0_hbm, out1_hbm, deg0_hbm, deg1_hbm,
                   src_v, dst_v, rows_v, deg_v, rowidx, acc, dacc,
                   sem_g0, sem_g1, sem_g2, sem_s0, sem_s1, sem_s2):
        sem_g = (sem_g0, sem_g1, sem_g2)
        sem_s = (sem_s0, sem_s1, sem_s2)
        cid = lax.axis_index("c")
        sid = lax.axis_index("s")
        wid = sid * NCU + cid

        z16 = jnp.zeros((16,), jnp.float32)

        def zdeg(i, carry):
            for j in range(128 // 16):
                deg_v[i, pl.ds(j * 16, 16)] = z16
            return carry

        lax.fori_loop(0, dr, zdeg, 0)
        for m in range(dr // 16):
            rowidx[pl.ds(m * 16, 16)] = lax.iota(jnp.int32, 16) + m * 16

        pltpu.sync_copy(zero_hbm.at[pl.ds(0, rpt)], acc.at[pl.ds(sid * rpt, rpt)])

        @pl.when(sid < dr // 8)
        def _():
            pltpu.sync_copy(zero_hbm.at[pl.ds(0, 8)], dacc.at[pl.ds(sid * 8, 8)])

        plsc.subcore_barrier()

        def block(bi, carry):
            pltpu.sync_copy(edge_hbm.at[0, wid * nb + bi], src_v)
            pltpu.sync_copy(edge_hbm.at[1, wid * nb + bi], dst_v)

            nbuf = 2
            gathers = [None] * nbuf
            scatters = [None] * nbuf
            for j0 in range(nbuf - 1):
                gathers[j0] = pltpu.async_copy(y_hbm.at[src_v.at[j0]],
                                               rows_v.at[j0], sem_g[j0])
            for j in range(CPB):
                p = j % nbuf
                q = (j + nbuf - 1) % nbuf
                if j + nbuf - 1 < CPB:
                    if scatters[q] is not None:
                        scatters[q].wait()
                        scatters[q] = None
                    gathers[q] = pltpu.async_copy(
                        y_hbm.at[src_v.at[j + nbuf - 1]],
                        rows_v.at[q], sem_g[q])
                gathers[p].wait()
                scatters[p] = pltpu.async_copy(rows_v.at[p],
                                               acc.at[dst_v.at[j]],
                                               sem_s[p], add=True)
                for k in range(CK // 16):
                    vec = dst_v[j, pl.ds(k * 16, 16)]
                    cnt, last = plsc.scan_count(vec)
                    plsc.addupdate_scatter(deg_v,
                                           [lax.shift_right_logical(vec, 7),
                                            lax.bitwise_and(vec, 127)],
                                           cnt.astype(jnp.float32), mask=last)
            for s in scatters:
                if s is not None:
                    s.wait()
            return carry

        lax.fori_loop(0, nb, block, 0)

        pltpu.sync_copy(deg_v, dacc.at[rowidx], add=True)
        plsc.subcore_barrier()

        @pl.when(cid == 0)
        def _():
            pltpu.sync_copy(acc.at[pl.ds(sid * rpt, rpt)],
                            out0_hbm.at[pl.ds(sid * rpt, rpt)])

            @pl.when(sid < dr // 8)
            def _():
                pltpu.sync_copy(dacc.at[pl.ds(sid * 8, 8)],
                                deg0_hbm.at[pl.ds(sid * 8, 8)])

        @pl.when(cid == 1)
        def _():
            pltpu.sync_copy(acc.at[pl.ds(sid * rpt, rpt)],
                            out1_hbm.at[pl.ds(sid * rpt, rpt)])

            @pl.when(sid < dr // 8)
            def _():
                pltpu.sync_copy(dacc.at[pl.ds(sid * 8, 8)],
                                deg1_hbm.at[pl.ds(sid * 8, 8)])

    return sc_scatter, n_pad, dr


def kernel(x, edge_index, W_l, W_r, b):
    n, d = x.shape
    e = edge_index.shape[1]
    blk = 1024

    bsz = CPB * CK
    epw = -(-(e // NW) // bsz) * bsz
    sc_scatter, n_pad, dr = _make_sc_scatter(n, NW * epw, d)
    pad = NW * epw - e
    edges = edge_index
    if pad:
        ar = jnp.arange(pad, dtype=jnp.int32)
        edges = jnp.concatenate(
            [edge_index,
             jnp.stack([(ar * 977) % n, n + ar % (n_pad - n)])], axis=1)
    edge4 = edges.reshape(2, -1, CPB, CK)
    zeros = jnp.zeros((n_pad // NS, d), jnp.float32)
    acc0, acc1, deg0, deg1 = sc_scatter(x, edge4, zeros)

    out = pl.pallas_call(
        _combine_body,
        grid=(-(-n // blk),),
        in_specs=[pl.BlockSpec((blk, d), lambda i: (i, 0)),
                  pl.BlockSpec((blk, d), lambda i: (i, 0)),
                  pl.BlockSpec((blk // 128, 128), lambda i: (i, 0)),
                  pl.BlockSpec((blk // 128, 128), lambda i: (i, 0)),
                  pl.BlockSpec((blk, d), lambda i: (i, 0)),
                  pl.BlockSpec((d, d), lambda i: (0, 0)),
                  pl.BlockSpec((d, d), lambda i: (0, 0)),
                  pl.BlockSpec((1, d), lambda i: (0, 0))],
        out_specs=pl.BlockSpec((blk, d), lambda i: (i, 0)),
        out_shape=jax.ShapeDtypeStruct((n, d), jnp.float32),
    )(acc0, acc1, deg0, deg1, x, W_l, W_r, b.reshape(1, d))

    return out
